# baseline (speedup 1.0000x reference)
import functools

import jax
import jax.numpy as jnp
from jax import lax
from jax.experimental import pallas as pl
from jax.experimental.pallas import tpu as pltpu
from jax.experimental.pallas import tpu_sc as plsc

N = 10000
E = 320000
D = 128
NC = 2
NS = 16
NW = NC * NS
CHUNK = 128
NCHUNK = 80
E_PAD = NW * CHUNK * NCHUNK
N_ACC = 10240
ZROWS = 16
RPS_ACC = N_ACC // NS
BLK = 1000

_sc_mesh = functools.partial(
    plsc.VectorSubcoreMesh, core_axis_name="c", subcore_axis_name="s"
)


def _degree_sc(dst_r):
    @functools.partial(
        pl.kernel,
        mesh=_sc_mesh(),
        out_type=jax.ShapeDtypeStruct((NC, N_ACC, D), jnp.float32),
        scratch_types=[
            pltpu.VMEM((NCHUNK, CHUNK), jnp.int32),
            pltpu.VMEM((CHUNK, D), jnp.float32),
            pltpu.VMEM((ZROWS, D), jnp.float32),
            pltpu.VMEM_SHARED((N_ACC, D), jnp.float32),
        ],
    )
    def k(dst_hbm, out_hbm, idx_v, ones_v, z_v, acc_sh):
        c = lax.axis_index("c")
        s = lax.axis_index("s")
        wid = c * NS + s

        @pl.loop(0, CHUNK)
        def _(i):
            for kk in range(D // 16):
                ones_v[i, pl.ds(kk * 16, 16)] = jnp.ones((16,), jnp.float32)

        @pl.loop(0, ZROWS)
        def _(i):
            for kk in range(D // 16):
                z_v[i, pl.ds(kk * 16, 16)] = jnp.zeros((16,), jnp.float32)

        @pl.loop(0, RPS_ACC // ZROWS)
        def _(i):
            pltpu.sync_copy(
                z_v, acc_sh.at[pl.ds(s * RPS_ACC + i * ZROWS, ZROWS)]
            )

        pltpu.sync_copy(dst_hbm.at[wid], idx_v)
        plsc.subcore_barrier()

        @pl.loop(0, NCHUNK)
        def _(j):
            pltpu.sync_copy(ones_v, acc_sh.at[idx_v.at[j]], add=True)

        plsc.subcore_barrier()
        sl = pl.ds(s * RPS_ACC, RPS_ACC)
        pltpu.sync_copy(acc_sh.at[sl], out_hbm.at[c].at[sl])

    return k(dst_r)


NBUF = 2
DH = D // 2
RPS_TAB = N_ACC // NS
NSTG = 2
CPS = NCHUNK // NSTG


def _edge_aggregate_sc(table, src_r, dst_r):
    @functools.partial(
        pl.kernel,
        mesh=_sc_mesh(),
        out_type=jax.ShapeDtypeStruct((NC, N_ACC, D), jnp.float32),
        scratch_types=[
            pltpu.VMEM((NCHUNK, CHUNK), jnp.int32),
            pltpu.VMEM((NCHUNK, CHUNK), jnp.int32),
            pltpu.VMEM((CHUNK, D), jnp.float32),
            pltpu.VMEM((ZROWS, D), jnp.float32),
            pltpu.VMEM_SHARED((N_ACC, D), jnp.float32),
            pltpu.SemaphoreType.DMA,
        ],
    )
    def k(tab_hbm, src_hbm, dst_hbm, out_hbm, si_v, di_v, rows_v, z_v,
          acc_sh, sem):
        c = lax.axis_index("c")
        s = lax.axis_index("s")
        wid = c * NS + s

        @pl.loop(0, ZROWS)
        def _(i):
            for kk in range(D // 16):
                z_v[i, pl.ds(kk * 16, 16)] = jnp.zeros((16,), jnp.float32)

        @pl.loop(0, RPS_ACC // ZROWS)
        def _(i):
            pltpu.sync_copy(
                z_v, acc_sh.at[pl.ds(s * RPS_ACC + i * ZROWS, ZROWS)]
            )

        pltpu.sync_copy(src_hbm.at[wid], si_v)
        pltpu.sync_copy(dst_hbm.at[wid], di_v)
        plsc.subcore_barrier()

        @pl.loop(0, NCHUNK)
        def _(j):
            pltpu.async_copy(tab_hbm.at[si_v.at[j]], rows_v, sem).wait()
            pltpu.sync_copy(rows_v, acc_sh.at[di_v.at[j]], add=True)

        plsc.subcore_barrier()
        sl = pl.ds(s * RPS_ACC, RPS_ACC)
        pltpu.sync_copy(acc_sh.at[sl], out_hbm.at[c].at[sl])

    return k(table, src_r, dst_r)


def _row_spec():
    return pl.BlockSpec((BLK, D), lambda i: (i, 0))


def _full_spec(shape):
    nd = len(shape)
    return pl.BlockSpec(shape, lambda i: (0,) * nd)


def _deg_spec():
    return pl.BlockSpec((NC, BLK, D), lambda i: (0, i, 0))


def _acc_spec():
    return pl.BlockSpec((NC, BLK, D), lambda i: (0, i, 0))


def _dinv_block(degp):
    deg = degp[0, :, 0] + degp[1, :, 0] + 1.0
    return lax.rsqrt(deg)[:, None]


def _mm_body(x_ref, w1_ref, wfc_ref, bfc_ref, h1_ref, res_ref):
    xb = x_ref[...]
    h1_ref[...] = jnp.dot(xb, w1_ref[...], preferred_element_type=jnp.float32)
    res_ref[...] = (
        jnp.dot(xb, wfc_ref[...], preferred_element_type=jnp.float32)
        + bfc_ref[...]
    )


def _mm_tc(x, W1, Wfc, bfc):
    out_sh = jax.ShapeDtypeStruct((N, D), jnp.float32)
    return pl.pallas_call(
        _mm_body,
        grid=(N // BLK,),
        in_specs=[
            _row_spec(),
            _full_spec((D, D)),
            _full_spec((D, D)),
            _full_spec((D,)),
        ],
        out_specs=[_row_spec(), _row_spec()],
        out_shape=[out_sh, out_sh],
    )(x, W1, Wfc, bfc)


def _scale_body(h_ref, degp_ref, hp_ref):
    hp_ref[...] = h_ref[...] * _dinv_block(degp_ref[...])


def _scale_tc(h, degp):
    return pl.pallas_call(
        _scale_body,
        grid=(N // BLK,),
        in_specs=[_row_spec(), _deg_spec()],
        out_specs=_row_spec(),
        out_shape=jax.ShapeDtypeStruct((N, D), jnp.float32),
    )(h, degp)


def _mid_body(acc_ref, hp1_ref, degp_ref, b1_ref, w2_ref, hp2_ref):
    dinv = _dinv_block(degp_ref[...])
    acc = acc_ref[...]
    h1 = jax.nn.relu(
        dinv * (acc[0] + acc[1] + hp1_ref[...]) + b1_ref[...]
    )
    hp2_ref[...] = dinv * jnp.dot(
        h1, w2_ref[...], preferred_element_type=jnp.float32
    )


def _mid_tc(acc1, hp1, degp, b1, W2):
    return pl.pallas_call(
        _mid_body,
        grid=(N // BLK,),
        in_specs=[
            _acc_spec(),
            _row_spec(),
            _deg_spec(),
            _full_spec((D,)),
            _full_spec((D, D)),
        ],
        out_specs=_row_spec(),
        out_shape=jax.ShapeDtypeStruct((N, D), jnp.float32),
    )(acc1, hp1, degp, b1, W2)


def _post_body(acc_ref, hp2_ref, degp_ref, b2_ref, res_ref, out_ref):
    dinv = _dinv_block(degp_ref[...])
    acc = acc_ref[...]
    out_ref[...] = jax.nn.relu(
        dinv * (acc[0] + acc[1] + hp2_ref[...]) + b2_ref[...] + res_ref[...]
    )


def _post_tc(acc2, hp2, degp, b2, res):
    return pl.pallas_call(
        _post_body,
        grid=(N // BLK,),
        in_specs=[
            _acc_spec(),
            _row_spec(),
            _deg_spec(),
            _full_spec((D,)),
            _row_spec(),
        ],
        out_specs=_row_spec(),
        out_shape=jax.ShapeDtypeStruct((N, D), jnp.float32),
    )(acc2, hp2, degp, b2, res)


def kernel(x, edge_index, W1, b1, W2, b2, Wfc, bfc):
    pad = E_PAD - E
    src = jnp.concatenate(
        [edge_index[0].astype(jnp.int32), jnp.zeros((pad,), jnp.int32)]
    ).reshape(NW, NCHUNK, CHUNK)
    dst = jnp.concatenate(
        [edge_index[1].astype(jnp.int32), jnp.full((pad,), N, jnp.int32)]
    ).reshape(NW, NCHUNK, CHUNK)

    degp = _degree_sc(dst)
    h1, res = _mm_tc(x, W1, Wfc, bfc)
    hp1 = _scale_tc(h1, degp)
    acc1 = _edge_aggregate_sc(hp1, src, dst)
    hp2 = _mid_tc(acc1, hp1, degp, b1, W2)
    acc2 = _edge_aggregate_sc(hp2, src, dst)
    return _post_tc(acc2, hp2, degp, b2, res)


# speedup vs baseline: 1.4856x; 1.4856x over previous
import functools

import jax
import jax.numpy as jnp
from jax import lax
from jax.experimental import pallas as pl
from jax.experimental.pallas import tpu as pltpu
from jax.experimental.pallas import tpu_sc as plsc

N = 10000
E = 320000
D = 128
NC = 2
NS = 16
NW = NC * NS
CHUNK = 128
NCHUNK = 79
E_PAD = NW * CHUNK * NCHUNK
N_ACC = 10240
ZROWS = 16
RPS_ACC = N_ACC // NS
BLK = 1000

_sc_mesh = functools.partial(
    plsc.VectorSubcoreMesh, core_axis_name="c", subcore_axis_name="s"
)


def _degree_sc(dst_r):
    @functools.partial(
        pl.kernel,
        mesh=_sc_mesh(),
        out_type=jax.ShapeDtypeStruct((NC, N_ACC, D), jnp.float32),
        scratch_types=[
            pltpu.VMEM((NCHUNK, CHUNK), jnp.int32),
            pltpu.VMEM((CHUNK, D), jnp.float32),
            pltpu.VMEM((ZROWS, D), jnp.float32),
            pltpu.VMEM_SHARED((N_ACC, D), jnp.float32),
        ],
    )
    def k(dst_hbm, out_hbm, idx_v, ones_v, z_v, acc_sh):
        c = lax.axis_index("c")
        s = lax.axis_index("s")
        wid = c * NS + s

        @pl.loop(0, CHUNK)
        def _(i):
            for kk in range(D // 16):
                ones_v[i, pl.ds(kk * 16, 16)] = jnp.ones((16,), jnp.float32)

        @pl.loop(0, ZROWS)
        def _(i):
            for kk in range(D // 16):
                z_v[i, pl.ds(kk * 16, 16)] = jnp.zeros((16,), jnp.float32)

        @pl.loop(0, RPS_ACC // ZROWS)
        def _(i):
            pltpu.sync_copy(
                z_v, acc_sh.at[pl.ds(s * RPS_ACC + i * ZROWS, ZROWS)]
            )

        pltpu.sync_copy(dst_hbm.at[wid], idx_v)
        plsc.subcore_barrier()

        @pl.loop(0, NCHUNK)
        def _(j):
            pltpu.sync_copy(ones_v, acc_sh.at[idx_v.at[j]], add=True)

        plsc.subcore_barrier()
        sl = pl.ds(s * RPS_ACC, RPS_ACC)
        pltpu.sync_copy(acc_sh.at[sl], out_hbm.at[c].at[sl])

    return k(dst_r)


NBUF = 2
DH = D // 2
RPS_TAB = N_ACC // NS
NSTG = 2
CPS = NCHUNK // NSTG


def _edge_aggregate_sc(table, src_r, dst_r):
    @functools.partial(
        pl.kernel,
        mesh=_sc_mesh(),
        out_type=jax.ShapeDtypeStruct((NC, N_ACC, D), jnp.float32),
        scratch_types=[
            pltpu.VMEM((NCHUNK, CHUNK), jnp.int32),
            pltpu.VMEM((NCHUNK, CHUNK), jnp.int32),
            pltpu.VMEM((CHUNK, D), jnp.float32),
            pltpu.VMEM((ZROWS, D), jnp.float32),
            pltpu.VMEM_SHARED((N_ACC, D), jnp.float32),
            pltpu.SemaphoreType.DMA,
        ],
    )
    def k(tab_hbm, src_hbm, dst_hbm, out_hbm, si_v, di_v, rows_v, z_v,
          acc_sh, sem):
        c = lax.axis_index("c")
        s = lax.axis_index("s")
        wid = c * NS + s

        @pl.loop(0, ZROWS)
        def _(i):
            for kk in range(D // 16):
                z_v[i, pl.ds(kk * 16, 16)] = jnp.zeros((16,), jnp.float32)

        @pl.loop(0, RPS_ACC // ZROWS)
        def _(i):
            pltpu.sync_copy(
                z_v, acc_sh.at[pl.ds(s * RPS_ACC + i * ZROWS, ZROWS)]
            )

        pltpu.sync_copy(src_hbm.at[wid], si_v)
        pltpu.sync_copy(dst_hbm.at[wid], di_v)
        plsc.subcore_barrier()

        @pl.loop(0, NCHUNK)
        def _(j):
            pltpu.async_copy(tab_hbm.at[si_v.at[j]], rows_v, sem).wait()
            pltpu.sync_copy(rows_v, acc_sh.at[di_v.at[j]], add=True)

        plsc.subcore_barrier()
        sl = pl.ds(s * RPS_ACC, RPS_ACC)
        pltpu.sync_copy(acc_sh.at[sl], out_hbm.at[c].at[sl])

    return k(table, src_r, dst_r)


def _row_spec():
    return pl.BlockSpec((BLK, D), lambda i: (i, 0))


def _full_spec(shape):
    nd = len(shape)
    return pl.BlockSpec(shape, lambda i: (0,) * nd)


def _deg_spec():
    return pl.BlockSpec((NC, BLK, D), lambda i: (0, i, 0))


def _acc_spec():
    return pl.BlockSpec((NC, BLK, D), lambda i: (0, i, 0))


def _dinv_block(degp):
    deg = degp[0, :, 0] + degp[1, :, 0] + 1.0
    return lax.rsqrt(deg)[:, None]


def _mm_body(x_ref, w1_ref, wfc_ref, bfc_ref, h1_ref, res_ref):
    xb = x_ref[...]
    h1_ref[...] = jnp.dot(xb, w1_ref[...], preferred_element_type=jnp.float32)
    res_ref[...] = (
        jnp.dot(xb, wfc_ref[...], preferred_element_type=jnp.float32)
        + bfc_ref[...]
    )


def _mm_tc(x, W1, Wfc, bfc):
    out_sh = jax.ShapeDtypeStruct((N, D), jnp.float32)
    return pl.pallas_call(
        _mm_body,
        grid=(N // BLK,),
        in_specs=[
            _row_spec(),
            _full_spec((D, D)),
            _full_spec((D, D)),
            _full_spec((D,)),
        ],
        out_specs=[_row_spec(), _row_spec()],
        out_shape=[out_sh, out_sh],
    )(x, W1, Wfc, bfc)


def _scale_body(h_ref, degp_ref, hp_ref):
    hp_ref[...] = h_ref[...] * _dinv_block(degp_ref[...])


def _scale_tc(h, degp):
    return pl.pallas_call(
        _scale_body,
        grid=(N // BLK,),
        in_specs=[_row_spec(), _deg_spec()],
        out_specs=_row_spec(),
        out_shape=jax.ShapeDtypeStruct((N, D), jnp.float32),
    )(h, degp)


def _mid_body(acc_ref, hp1_ref, degp_ref, b1_ref, w2_ref, hp2_ref):
    dinv = _dinv_block(degp_ref[...])
    acc = acc_ref[...]
    h1 = jax.nn.relu(
        dinv * (acc[0] + acc[1] + hp1_ref[...]) + b1_ref[...]
    )
    hp2_ref[...] = dinv * jnp.dot(
        h1, w2_ref[...], preferred_element_type=jnp.float32
    )


def _mid_tc(acc1, hp1, degp, b1, W2):
    return pl.pallas_call(
        _mid_body,
        grid=(N // BLK,),
        in_specs=[
            _acc_spec(),
            _row_spec(),
            _deg_spec(),
            _full_spec((D,)),
            _full_spec((D, D)),
        ],
        out_specs=_row_spec(),
        out_shape=jax.ShapeDtypeStruct((N, D), jnp.float32),
    )(acc1, hp1, degp, b1, W2)


def _post_body(acc_ref, hp2_ref, degp_ref, b2_ref, res_ref, out_ref):
    dinv = _dinv_block(degp_ref[...])
    acc = acc_ref[...]
    out_ref[...] = jax.nn.relu(
        dinv * (acc[0] + acc[1] + hp2_ref[...]) + b2_ref[...] + res_ref[...]
    )


def _post_tc(acc2, hp2, degp, b2, res):
    return pl.pallas_call(
        _post_body,
        grid=(N // BLK,),
        in_specs=[
            _acc_spec(),
            _row_spec(),
            _deg_spec(),
            _full_spec((D,)),
            _row_spec(),
        ],
        out_specs=_row_spec(),
        out_shape=jax.ShapeDtypeStruct((N, D), jnp.float32),
    )(acc2, hp2, degp, b2, res)


def kernel(x, edge_index, W1, b1, W2, b2, Wfc, bfc):
    pad = E_PAD - E
    src = jnp.concatenate(
        [edge_index[0].astype(jnp.int32), jnp.zeros((pad,), jnp.int32)]
    ).reshape(NW, NCHUNK, CHUNK)
    pad_dst = N + jnp.arange(pad, dtype=jnp.int32) % (N_ACC - N)
    dst = jnp.concatenate(
        [edge_index[1].astype(jnp.int32), pad_dst]
    ).reshape(NW, NCHUNK, CHUNK)

    degp = _degree_sc(dst)
    h1, res = _mm_tc(x, W1, Wfc, bfc)
    hp1 = _scale_tc(h1, degp)
    acc1 = _edge_aggregate_sc(hp1, src, dst)
    hp2 = _mid_tc(acc1, hp1, degp, b1, W2)
    acc2 = _edge_aggregate_sc(hp2, src, dst)
    return _post_tc(acc2, hp2, degp, b2, res)


# speedup vs baseline: 2.4710x; 1.6633x over previous
import functools

import jax
import jax.numpy as jnp
from jax import lax
from jax.experimental import pallas as pl
from jax.experimental.pallas import tpu as pltpu
from jax.experimental.pallas import tpu_sc as plsc

N = 10000
E = 320000
D = 128
NC = 2
NS = 16
NW = NC * NS
CHUNK = 128
NCHUNK = 79
E_PAD = NW * CHUNK * NCHUNK
N_ACC = 10240
ZROWS = 16
RPS_ACC = N_ACC // NS
BLK = 1000

_sc_mesh = functools.partial(
    plsc.VectorSubcoreMesh, core_axis_name="c", subcore_axis_name="s"
)


def _degree_sc(dst_r):
    @functools.partial(
        pl.kernel,
        mesh=_sc_mesh(),
        out_type=jax.ShapeDtypeStruct((NC, N_ACC, D), jnp.float32),
        scratch_types=[
            pltpu.VMEM((NCHUNK, CHUNK), jnp.int32),
            pltpu.VMEM((CHUNK, D), jnp.float32),
            pltpu.VMEM((ZROWS, D), jnp.float32),
            pltpu.VMEM_SHARED((N_ACC, D), jnp.float32),
        ],
    )
    def k(dst_hbm, out_hbm, idx_v, ones_v, z_v, acc_sh):
        c = lax.axis_index("c")
        s = lax.axis_index("s")
        wid = c * NS + s

        @pl.loop(0, CHUNK)
        def _(i):
            for kk in range(D // 16):
                ones_v[i, pl.ds(kk * 16, 16)] = jnp.ones((16,), jnp.float32)

        @pl.loop(0, ZROWS)
        def _(i):
            for kk in range(D // 16):
                z_v[i, pl.ds(kk * 16, 16)] = jnp.zeros((16,), jnp.float32)

        @pl.loop(0, RPS_ACC // ZROWS)
        def _(i):
            pltpu.sync_copy(
                z_v, acc_sh.at[pl.ds(s * RPS_ACC + i * ZROWS, ZROWS)]
            )

        pltpu.sync_copy(dst_hbm.at[wid], idx_v)
        plsc.subcore_barrier()

        @pl.loop(0, NCHUNK)
        def _(j):
            pltpu.sync_copy(ones_v, acc_sh.at[idx_v.at[j]], add=True)

        plsc.subcore_barrier()
        sl = pl.ds(s * RPS_ACC, RPS_ACC)
        pltpu.sync_copy(acc_sh.at[sl], out_hbm.at[c].at[sl])

    return k(dst_r)


NBUF = 2
DH = D // 2
RPS_TAB = N_ACC // NS
NSTG = 2
CPS = NCHUNK // NSTG


def _edge_aggregate_sc(table, src_r, dst_r):
    @functools.partial(
        pl.kernel,
        mesh=_sc_mesh(),
        out_type=jax.ShapeDtypeStruct((NC, N_ACC, D), jnp.float32),
        scratch_types=[
            pltpu.VMEM((NCHUNK, CHUNK), jnp.int32),
            pltpu.VMEM((NCHUNK, CHUNK), jnp.int32),
            pltpu.VMEM((CHUNK, D), jnp.float32),
            pltpu.VMEM((ZROWS, D), jnp.float32),
            pltpu.VMEM_SHARED((N_ACC, D), jnp.float32),
            pltpu.SemaphoreType.DMA,
        ],
    )
    def k(tab_hbm, src_hbm, dst_hbm, out_hbm, si_v, di_v, rows_v, z_v,
          acc_sh, sem):
        c = lax.axis_index("c")
        s = lax.axis_index("s")
        wid = c * NS + s

        @pl.loop(0, ZROWS)
        def _(i):
            for kk in range(D // 16):
                z_v[i, pl.ds(kk * 16, 16)] = jnp.zeros((16,), jnp.float32)

        @pl.loop(0, RPS_ACC // ZROWS)
        def _(i):
            pltpu.sync_copy(
                z_v, acc_sh.at[pl.ds(s * RPS_ACC + i * ZROWS, ZROWS)]
            )

        pltpu.sync_copy(src_hbm.at[wid], si_v)
        pltpu.sync_copy(dst_hbm.at[wid], di_v)
        plsc.subcore_barrier()

        @pl.loop(0, NCHUNK)
        def _(j):
            pltpu.async_copy(tab_hbm.at[si_v.at[j]], rows_v, sem).wait()
            pltpu.sync_copy(rows_v, acc_sh.at[di_v.at[j]], add=True)

        plsc.subcore_barrier()
        sl = pl.ds(s * RPS_ACC, RPS_ACC)
        pltpu.sync_copy(acc_sh.at[sl], out_hbm.at[c].at[sl])

    return k(table, src_r, dst_r)


def _row_spec():
    return pl.BlockSpec((BLK, D), lambda i: (i, 0))


def _full_spec(shape):
    nd = len(shape)
    return pl.BlockSpec(shape, lambda i: (0,) * nd)


def _deg_spec():
    return pl.BlockSpec((NC, BLK, D), lambda i: (0, i, 0))


def _acc_spec():
    return pl.BlockSpec((NC, BLK, D), lambda i: (0, i, 0))


def _dinv_block(degp):
    deg = degp[0, :, 0] + degp[1, :, 0] + 1.0
    return lax.rsqrt(deg)[:, None]


def _mm_body(x_ref, w1_ref, wfc_ref, bfc_ref, h1_ref, res_ref):
    xb = x_ref[...]
    h1_ref[...] = jnp.dot(xb, w1_ref[...], preferred_element_type=jnp.float32)
    res_ref[...] = (
        jnp.dot(xb, wfc_ref[...], preferred_element_type=jnp.float32)
        + bfc_ref[...]
    )


def _mm_tc(x, W1, Wfc, bfc):
    out_sh = jax.ShapeDtypeStruct((N, D), jnp.float32)
    return pl.pallas_call(
        _mm_body,
        grid=(N // BLK,),
        in_specs=[
            _row_spec(),
            _full_spec((D, D)),
            _full_spec((D, D)),
            _full_spec((D,)),
        ],
        out_specs=[_row_spec(), _row_spec()],
        out_shape=[out_sh, out_sh],
    )(x, W1, Wfc, bfc)


def _scale_body(h_ref, degp_ref, hp_ref):
    hp_ref[...] = h_ref[...] * _dinv_block(degp_ref[...])


def _scale_tc(h, degp):
    return pl.pallas_call(
        _scale_body,
        grid=(N // BLK,),
        in_specs=[_row_spec(), _deg_spec()],
        out_specs=_row_spec(),
        out_shape=jax.ShapeDtypeStruct((N, D), jnp.float32),
    )(h, degp)


def _mid_body(acc_ref, hp1_ref, degp_ref, b1_ref, w2_ref, hp2_ref):
    dinv = _dinv_block(degp_ref[...])
    acc = acc_ref[...]
    h1 = jax.nn.relu(
        dinv * (acc[0] + acc[1] + hp1_ref[...]) + b1_ref[...]
    )
    hp2_ref[...] = dinv * jnp.dot(
        h1, w2_ref[...], preferred_element_type=jnp.float32
    )


def _mid_tc(acc1, hp1, degp, b1, W2):
    return pl.pallas_call(
        _mid_body,
        grid=(N // BLK,),
        in_specs=[
            _acc_spec(),
            _row_spec(),
            _deg_spec(),
            _full_spec((D,)),
            _full_spec((D, D)),
        ],
        out_specs=_row_spec(),
        out_shape=jax.ShapeDtypeStruct((N, D), jnp.float32),
    )(acc1, hp1, degp, b1, W2)


def _post_body(acc_ref, hp2_ref, degp_ref, b2_ref, res_ref, out_ref):
    dinv = _dinv_block(degp_ref[...])
    acc = acc_ref[...]
    out_ref[...] = jax.nn.relu(
        dinv * (acc[0] + acc[1] + hp2_ref[...]) + b2_ref[...] + res_ref[...]
    )


def _post_tc(acc2, hp2, degp, b2, res):
    return pl.pallas_call(
        _post_body,
        grid=(N // BLK,),
        in_specs=[
            _acc_spec(),
            _row_spec(),
            _deg_spec(),
            _full_spec((D,)),
            _row_spec(),
        ],
        out_specs=_row_spec(),
        out_shape=jax.ShapeDtypeStruct((N, D), jnp.float32),
    )(acc2, hp2, degp, b2, res)


def kernel(x, edge_index, W1, b1, W2, b2, Wfc, bfc):
    pad = E_PAD - E
    pad_src = jnp.arange(pad, dtype=jnp.int32) % N
    src = jnp.concatenate(
        [edge_index[0].astype(jnp.int32), pad_src]
    ).reshape(NW, NCHUNK, CHUNK)
    pad_dst = N + jnp.arange(pad, dtype=jnp.int32) % (N_ACC - N)
    dst = jnp.concatenate(
        [edge_index[1].astype(jnp.int32), pad_dst]
    ).reshape(NW, NCHUNK, CHUNK)

    degp = _degree_sc(dst)
    h1, res = _mm_tc(x, W1, Wfc, bfc)
    hp1 = _scale_tc(h1, degp)
    acc1 = _edge_aggregate_sc(hp1, src, dst)
    hp2 = _mid_tc(acc1, hp1, degp, b1, W2)
    acc2 = _edge_aggregate_sc(hp2, src, dst)
    return _post_tc(acc2, hp2, degp, b2, res)
